# baseline (speedup 1.0000x reference)
import jax
import jax.numpy as jnp
from jax.experimental import pallas as pl


def _embed_sum_block(x_ref, wt_ref, o_ref):
    wt = wt_ref[...]
    for i in range(x_ref.shape[0]):
        o_ref[i] = jnp.dot(wt, x_ref[i], preferred_element_type=jnp.float32)


def kernel(inputs, embedding_matrix):
    batch, timesteps, n_codes = inputs.shape
    embed_size = embedding_matrix.shape[1]

    xt = jnp.transpose(inputs, (0, 2, 1))
    wt = jnp.transpose(embedding_matrix, (1, 0))

    block_b = 16
    assert batch % block_b == 0
    grid = (batch // block_b,)

    out_t = pl.pallas_call(
        _embed_sum_block,
        grid=grid,
        in_specs=[
            pl.BlockSpec((block_b, n_codes, timesteps), lambda i: (i, 0, 0)),
            pl.BlockSpec((embed_size, n_codes), lambda i: (0, 0)),
        ],
        out_specs=pl.BlockSpec((block_b, embed_size, timesteps),
                               lambda i: (i, 0, 0)),
        out_shape=jax.ShapeDtypeStruct((batch, embed_size, timesteps),
                                       jnp.float32),
    )(xt, wt)

    return jnp.transpose(out_t, (0, 2, 1))


# speedup vs baseline: 10.0715x; 10.0715x over previous
import jax
import jax.numpy as jnp
from jax.experimental import pallas as pl


def _embed_sum_block(x_ref, wt_ref, o_ref):
    o_ref[0] = jnp.dot(wt_ref[...], x_ref[0],
                       preferred_element_type=jnp.float32)


def kernel(inputs, embedding_matrix):
    batch, timesteps, n_codes = inputs.shape
    embed_size = embedding_matrix.shape[1]

    x_tcb = jnp.transpose(inputs, (1, 2, 0))
    wt = jnp.transpose(embedding_matrix, (1, 0))

    block_n = 2048 if batch % 2048 == 0 else batch
    grid = (timesteps, batch // block_n)

    out_teb = pl.pallas_call(
        _embed_sum_block,
        grid=grid,
        in_specs=[
            pl.BlockSpec((1, n_codes, block_n), lambda t, j: (t, 0, j)),
            pl.BlockSpec((embed_size, n_codes), lambda t, j: (0, 0)),
        ],
        out_specs=pl.BlockSpec((1, embed_size, block_n),
                               lambda t, j: (t, 0, j)),
        out_shape=jax.ShapeDtypeStruct((timesteps, embed_size, batch),
                                       jnp.float32),
    )(x_tcb, wt)

    return jnp.transpose(out_teb, (2, 0, 1))
